# baseline (device time: 19604 ns/iter reference)
import functools

import jax
import jax.numpy as jnp
from jax import lax
from jax.experimental import pallas as pl
from jax.experimental.pallas import tpu as pltpu

N_DEV = 8
B, SQ, D = 2, 128, 512
H_PER = 8
DH = 64
KV_PER = 4


def kernel(x, Wq, Wo, K_ext, V_ext):
    my_i = lax.axis_index("i")
    k2 = lax.dynamic_slice_in_dim(K_ext, 2 * my_i, 2, axis=2)
    v2 = lax.dynamic_slice_in_dim(V_ext, 2 * my_i, 2, axis=2)
    kt = jnp.transpose(k2, (0, 2, 1, 3)).astype(jnp.bfloat16)
    vt = jnp.transpose(v2, (0, 2, 1, 3)).astype(jnp.bfloat16)

    def body(x_ref, wq_ref, wo_ref, k_ref, v_ref, out_ref,
             o_scr, acc_ref, rbuf, send_sems, recv_sems):
        my = lax.axis_index("i")
        partners = (
            my + 1 - 2 * lax.rem(my, 2),
            my + 3 - 2 * lax.rem(my, 4),
            lax.rem(my + 4, N_DEV),
        )

        barrier_sem = pltpu.get_barrier_semaphore()
        for p in partners:
            pl.semaphore_signal(
                barrier_sem, inc=1,
                device_id=(p,), device_id_type=pl.DeviceIdType.MESH,
            )
        pl.semaphore_wait(barrier_sem, 3)

        HALF = D // 2
        orders = ((0, 1, 2), (1, 2, 0), (2, 0, 1), (1, 0, 2))
        handles = [[None] * 4 for _ in range(3)]

        def start(step, q):
            p = partners[orders[q][step]]
            r = pltpu.make_async_remote_copy(
                src_ref=acc_ref.at[q],
                dst_ref=rbuf.at[step, q],
                send_sem=send_sems.at[step, q],
                recv_sem=recv_sems.at[step, q],
                device_id=(p,),
                device_id_type=pl.DeviceIdType.MESH,
            )
            r.start()
            handles[step][q] = r

        zt = jnp.zeros((SQ, DH), jnp.bfloat16)

        def blockdiag(t):
            return jnp.concatenate(
                [jnp.concatenate(
                    [t if hh == h else zt for hh in range(4)], axis=1)
                 for h in range(4)], axis=0)

        wqv = wq_ref[...].astype(jnp.bfloat16)
        wov = wo_ref[...].astype(jnp.bfloat16)
        for b in range(B):
            qb = lax.dot(x_ref[b].astype(jnp.bfloat16), wqv,
                         preferred_element_type=jnp.float32) * 0.125
            for gl in range(2):
                kh = k_ref[b, gl]
                vh = v_ref[b, gl]
                bdk = blockdiag(kh)
                bdv = blockdiag(vh)
                qs = qb[:, gl * 256:(gl + 1) * 256].astype(jnp.bfloat16)
                s = lax.dot_general(
                    qs, bdk, (((1,), (1,)), ((), ())),
                    preferred_element_type=jnp.float32)
                ps = []
                for h in range(4):
                    sh = s[:, h * SQ:(h + 1) * SQ]
                    p = jnp.exp(sh)
                    l = jnp.sum(p, axis=1, keepdims=True)
                    ps.append((p / l).astype(jnp.bfloat16))
                pw = jnp.concatenate(ps, axis=1)
                ow = lax.dot(pw, bdv,
                             preferred_element_type=jnp.float32)
                o_scr[b * SQ:(b + 1) * SQ, gl * 256:(gl + 1) * 256] = (
                    ow.astype(jnp.bfloat16)
                )
            for c in range(2):
                q = 2 * b + c
                acc_ref[q] = lax.dot(
                    o_scr[b * SQ:(b + 1) * SQ, :],
                    wov[:, c * HALF:(c + 1) * HALF],
                    preferred_element_type=jnp.float32,
                ).astype(jnp.bfloat16)
                start(0, q)

        for step in range(3):
            for q in range(4):
                handles[step][q].wait()
                acc_ref[q] = acc_ref[q] + rbuf[step, q]
                if step < 2:
                    start(step + 1, q)
                else:
                    b, c = q // 2, q % 2
                    out_ref[b, :, c * HALF:(c + 1) * HALF] = acc_ref[q]

        @functools.partial(
            pl.run_scoped, exit_sem=pltpu.SemaphoreType.REGULAR
        )
        def _(exit_sem):
            for p in partners:
                pl.semaphore_signal(
                    exit_sem, inc=1,
                    device_id=(p,), device_id_type=pl.DeviceIdType.MESH,
                )
            pl.semaphore_wait(exit_sem, 3)

    return pl.pallas_call(
        body,
        out_shape=jax.ShapeDtypeStruct((B, SQ, D), jnp.bfloat16),
        in_specs=[pl.BlockSpec(memory_space=pltpu.VMEM)] * 5,
        out_specs=pl.BlockSpec(memory_space=pltpu.VMEM),
        scratch_shapes=[
            pltpu.VMEM((B * SQ, D), jnp.bfloat16),
            pltpu.VMEM((4, SQ, D // 2), jnp.bfloat16),
            pltpu.VMEM((3, 4, SQ, D // 2), jnp.bfloat16),
            pltpu.SemaphoreType.DMA((3, 4)),
            pltpu.SemaphoreType.DMA((3, 4)),
        ],
        compiler_params=pltpu.CompilerParams(collective_id=5),
    )(x, Wq, Wo, kt, vt)


# device time: 18866 ns/iter; 1.0391x vs baseline; 1.0391x over previous
import functools

import jax
import jax.numpy as jnp
from jax import lax
from jax.experimental import pallas as pl
from jax.experimental.pallas import tpu as pltpu

N_DEV = 8
B, SQ, D = 2, 128, 512
H_PER = 8
DH = 64
KV_PER = 4


def kernel(x, Wq, Wo, K_ext, V_ext):
    xb = x.astype(jnp.bfloat16)
    wq = Wq.astype(jnp.bfloat16)
    wo = Wo.astype(jnp.bfloat16)
    my_i = lax.axis_index("i")
    k2 = lax.dynamic_slice_in_dim(K_ext, 2 * my_i, 2, axis=2)
    v2 = lax.dynamic_slice_in_dim(V_ext, 2 * my_i, 2, axis=2)
    kt = jnp.transpose(k2, (0, 2, 1, 3)).astype(jnp.bfloat16)
    vt = jnp.transpose(v2, (0, 2, 1, 3)).astype(jnp.bfloat16)

    def body(x_ref, wq_ref, wo_ref, k_ref, v_ref, out_ref,
             o_scr, acc_ref, rbuf, send_sems, recv_sems):
        my = lax.axis_index("i")
        partners = (
            my + 1 - 2 * lax.rem(my, 2),
            my + 3 - 2 * lax.rem(my, 4),
            lax.rem(my + 4, N_DEV),
        )

        barrier_sem = pltpu.get_barrier_semaphore()
        for p in partners:
            pl.semaphore_signal(
                barrier_sem, inc=1,
                device_id=(p,), device_id_type=pl.DeviceIdType.MESH,
            )
        pl.semaphore_wait(barrier_sem, 3)

        HALF = D // 2
        orders = ((0, 1, 2), (1, 2, 0), (2, 0, 1), (1, 0, 2))
        handles = [[None] * 4 for _ in range(3)]

        def start(step, q):
            p = partners[orders[q][step]]
            r = pltpu.make_async_remote_copy(
                src_ref=acc_ref.at[q],
                dst_ref=rbuf.at[step, q],
                send_sem=send_sems.at[step, q],
                recv_sem=recv_sems.at[step, q],
                device_id=(p,),
                device_id_type=pl.DeviceIdType.MESH,
            )
            r.start()
            handles[step][q] = r

        zt = jnp.zeros((SQ, DH), jnp.bfloat16)

        def blockdiag(t):
            return jnp.concatenate(
                [jnp.concatenate(
                    [t if hh == h else zt for hh in range(4)], axis=1)
                 for h in range(4)], axis=0)

        for b in range(B):
            qb = lax.dot(x_ref[b], wq_ref[...],
                         preferred_element_type=jnp.float32) * 0.125
            for gl in range(2):
                kh = k_ref[b, gl]
                vh = v_ref[b, gl]
                bdk = blockdiag(kh)
                bdv = blockdiag(vh)
                qs = qb[:, gl * 256:(gl + 1) * 256].astype(jnp.bfloat16)
                s = lax.dot_general(
                    qs, bdk, (((1,), (1,)), ((), ())),
                    preferred_element_type=jnp.float32)
                ps = []
                for h in range(4):
                    sh = s[:, h * SQ:(h + 1) * SQ]
                    p = jnp.exp(sh)
                    l = jnp.sum(p, axis=1, keepdims=True)
                    ps.append((p / l).astype(jnp.bfloat16))
                pw = jnp.concatenate(ps, axis=1)
                ow = lax.dot(pw, bdv,
                             preferred_element_type=jnp.float32)
                o_scr[b * SQ:(b + 1) * SQ, gl * 256:(gl + 1) * 256] = (
                    ow.astype(jnp.bfloat16)
                )
            for c in range(2):
                q = 2 * b + c
                acc_ref[q] = lax.dot(
                    o_scr[b * SQ:(b + 1) * SQ, :],
                    wo_ref[:, c * HALF:(c + 1) * HALF],
                    preferred_element_type=jnp.float32,
                ).astype(jnp.bfloat16)
                start(0, q)

        for step in range(3):
            for q in range(4):
                handles[step][q].wait()
                acc_ref[q] = acc_ref[q] + rbuf[step, q]
                if step < 2:
                    start(step + 1, q)
                else:
                    b, c = q // 2, q % 2
                    out_ref[b, :, c * HALF:(c + 1) * HALF] = acc_ref[q]

        @functools.partial(
            pl.run_scoped, exit_sem=pltpu.SemaphoreType.REGULAR
        )
        def _(exit_sem):
            for p in partners:
                pl.semaphore_signal(
                    exit_sem, inc=1,
                    device_id=(p,), device_id_type=pl.DeviceIdType.MESH,
                )
            pl.semaphore_wait(exit_sem, 3)

    return pl.pallas_call(
        body,
        out_shape=jax.ShapeDtypeStruct((B, SQ, D), jnp.bfloat16),
        in_specs=[pl.BlockSpec(memory_space=pltpu.VMEM)] * 5,
        out_specs=pl.BlockSpec(memory_space=pltpu.VMEM),
        scratch_shapes=[
            pltpu.VMEM((B * SQ, D), jnp.bfloat16),
            pltpu.VMEM((4, SQ, D // 2), jnp.bfloat16),
            pltpu.VMEM((3, 4, SQ, D // 2), jnp.bfloat16),
            pltpu.SemaphoreType.DMA((3, 4)),
            pltpu.SemaphoreType.DMA((3, 4)),
        ],
        compiler_params=pltpu.CompilerParams(collective_id=5),
    )(xb, wq, wo, kt, vt)


# device time: 18842 ns/iter; 1.0404x vs baseline; 1.0013x over previous
import functools

import jax
import jax.numpy as jnp
from jax import lax
from jax.experimental import pallas as pl
from jax.experimental.pallas import tpu as pltpu

N_DEV = 8
B, SQ, D = 2, 128, 512
DH = 64


def kernel(x, Wq, Wo, K_ext, V_ext):
    xb = x.astype(jnp.bfloat16)
    wq = Wq.astype(jnp.bfloat16)
    wo = Wo.astype(jnp.bfloat16)
    my_i = lax.axis_index("i")
    k2 = lax.dynamic_slice_in_dim(K_ext, 2 * my_i, 2, axis=2)
    v2 = lax.dynamic_slice_in_dim(V_ext, 2 * my_i, 2, axis=2)
    kt = jnp.transpose(k2, (0, 2, 1, 3)).astype(jnp.bfloat16)
    vt = jnp.transpose(v2, (0, 2, 1, 3)).astype(jnp.bfloat16)

    def body(x_ref, wq_ref, wo_ref, k_ref, v_ref, out_ref,
             o_scr, acc_ref, rbuf, send_sems, recv_sems):
        my = lax.axis_index("i")
        partners = (
            my + 1 - 2 * lax.rem(my, 2),
            my + 3 - 2 * lax.rem(my, 4),
            lax.rem(my + 4, N_DEV),
        )

        barrier_sem = pltpu.get_barrier_semaphore()
        for p in partners:
            pl.semaphore_signal(
                barrier_sem, inc=1,
                device_id=(p,), device_id_type=pl.DeviceIdType.MESH,
            )
        pl.semaphore_wait(barrier_sem, 3)

        HALF = D // 2
        orders = ((0, 1, 2), (1, 2, 0), (2, 0, 1), (1, 0, 2))
        handles = [[None] * 4 for _ in range(3)]

        def start(step, q):
            p = partners[orders[q][step]]
            r = pltpu.make_async_remote_copy(
                src_ref=acc_ref.at[q],
                dst_ref=rbuf.at[step, q],
                send_sem=send_sems.at[step, q],
                recv_sem=recv_sems.at[step, q],
                device_id=(p,),
                device_id_type=pl.DeviceIdType.MESH,
            )
            r.start()
            handles[step][q] = r

        zt = jnp.zeros((SQ, DH), jnp.bfloat16)

        def blockdiag(t):
            return jnp.concatenate(
                [jnp.concatenate(
                    [t if hh == h else zt for hh in range(4)], axis=1)
                 for h in range(4)], axis=0)

        for b in range(B):
            qb = lax.dot(x_ref[b], wq_ref[...],
                         preferred_element_type=jnp.float32) * 0.125
            for gl in range(2):
                kh = k_ref[b, gl]
                vh = v_ref[b, gl]
                bdk = blockdiag(kh)
                bdv = blockdiag(vh)
                qs = qb[:, gl * 256:(gl + 1) * 256].astype(jnp.bfloat16)
                s = lax.dot_general(
                    qs, bdk, (((1,), (1,)), ((), ())),
                    preferred_element_type=jnp.float32)
                ps = []
                for h in range(4):
                    sh = s[:, h * SQ:(h + 1) * SQ]
                    p = jnp.exp(sh)
                    l = jnp.sum(p, axis=1, keepdims=True)
                    ps.append((p / l).astype(jnp.bfloat16))
                pw = jnp.concatenate(ps, axis=1)
                ow = lax.dot(pw, bdv,
                             preferred_element_type=jnp.float32)
                o_scr[b * SQ:(b + 1) * SQ, gl * 256:(gl + 1) * 256] = (
                    ow.astype(jnp.bfloat16)
                )
            for c in range(2):
                q = 2 * b + c
                acc_ref[q] = lax.dot(
                    o_scr[b * SQ:(b + 1) * SQ, :],
                    wo_ref[:, c * HALF:(c + 1) * HALF],
                    preferred_element_type=jnp.float32,
                ).astype(jnp.bfloat16)
                start(0, q)

        for step in range(3):
            for q in range(4):
                handles[step][q].wait()
                acc_ref[q] = acc_ref[q] + rbuf[step, q]
                if step < 2:
                    start(step + 1, q)
                else:
                    b, c = q // 2, q % 2
                    out_ref[b, :, c * HALF:(c + 1) * HALF] = acc_ref[q]

        @functools.partial(
            pl.run_scoped, exit_sem=pltpu.SemaphoreType.REGULAR
        )
        def _(exit_sem):
            for p in partners:
                pl.semaphore_signal(
                    exit_sem, inc=1,
                    device_id=(p,), device_id_type=pl.DeviceIdType.MESH,
                )
            pl.semaphore_wait(exit_sem, 3)

    return pl.pallas_call(
        body,
        out_shape=jax.ShapeDtypeStruct((B, SQ, D), jnp.bfloat16),
        in_specs=[pl.BlockSpec(memory_space=pltpu.VMEM)] * 5,
        out_specs=pl.BlockSpec(memory_space=pltpu.VMEM),
        scratch_shapes=[
            pltpu.VMEM((B * SQ, D), jnp.bfloat16),
            pltpu.VMEM((4, SQ, D // 2), jnp.bfloat16),
            pltpu.VMEM((3, 4, SQ, D // 2), jnp.bfloat16),
            pltpu.SemaphoreType.DMA((3, 4)),
            pltpu.SemaphoreType.DMA((3, 4)),
        ],
        compiler_params=pltpu.CompilerParams(collective_id=5),
    )(xb, wq, wo, kt, vt)
